# baseline (speedup 1.0000x reference)
import functools

import jax
import jax.numpy as jnp
from jax import lax
from jax.experimental import pallas as pl
from jax.experimental.pallas import tpu as pltpu
from jax.experimental.pallas import tpu_sc as plsc

_N = 10000
_D = 128
_H = 128
_C = 10
_G = 64
_E = 320000
_EPS = 1e-5

_NC = 2
_NS = 16
_NW = _NC * _NS
_CHUNK = 32
_NBUF = 8
_ACH = 320
_GRP = 40
_RCH = _E // _CHUNK

_NPAD = 10240
_RPT = _NPAD // _NS


_sc_mesh = dict(core_axis_name="c", subcore_axis_name="s")


def _split(c, s):
    base = (c * _NS + s) * _ACH
    ntail = (_RCH - (_NW - 1) * _ACH) // _GRP
    ngrp = jnp.where((c == 1) & (s == _NS - 1), ntail, _ACH // _GRP)
    return base, ngrp


@functools.partial(
    pl.kernel,
    out_type=jax.ShapeDtypeStruct((_NC, _NPAD), jnp.float32),
    mesh=plsc.VectorSubcoreMesh(**_sc_mesh),
    scratch_types=[
        pltpu.VMEM((_GRP, _CHUNK), jnp.int32),
        pltpu.VMEM((_CHUNK,), jnp.float32),
        pltpu.VMEM_SHARED((_NPAD,), jnp.float32),
    ],
)
def _sc_degree(cols_hbm, zeros1_hbm, out_hbm, cols_v, ones_v, deg_sh):
    c = lax.axis_index("c")
    s = lax.axis_index("s")
    base, ngrp = _split(c, s)
    for j in range(_CHUNK // 16):
        ones_v[pl.ds(j * 16, 16)] = jnp.full((16,), 1.0, jnp.float32)
    pltpu.sync_copy(zeros1_hbm.at[pl.ds(s * _RPT, _RPT)],
                    deg_sh.at[pl.ds(s * _RPT, _RPT)])
    plsc.subcore_barrier()

    def group(g, carry):
        pltpu.sync_copy(cols_hbm.at[pl.ds(base + g * _GRP, _GRP)], cols_v)

        def body(i, carry2):
            pltpu.sync_copy(ones_v, deg_sh.at[cols_v.at[i]], add=True)
            return carry2

        lax.fori_loop(0, _GRP, body, 0)
        return carry

    lax.fori_loop(0, ngrp, group, 0)
    plsc.subcore_barrier()
    pltpu.sync_copy(deg_sh.at[pl.ds(s * _RPT, _RPT)],
                    out_hbm.at[c, pl.ds(s * _RPT, _RPT)])


@functools.partial(
    pl.kernel,
    out_type=jax.ShapeDtypeStruct((_NC, _NPAD, _H), jnp.float32),
    mesh=plsc.VectorSubcoreMesh(**_sc_mesh),
    scratch_types=[
        pltpu.VMEM((_GRP, _CHUNK), jnp.int32),
        pltpu.VMEM((_GRP, _CHUNK), jnp.int32),
        pltpu.VMEM((_NBUF, _CHUNK, _H), jnp.float32),
        pltpu.VMEM((32, _H), jnp.float32),
        pltpu.VMEM_SHARED((_NPAD, _H), jnp.float32),
    ] + [pltpu.SemaphoreType.DMA] * 8,
)
def _sc_scatter(rows_hbm, cols_hbm, table_hbm, out_hbm,
                rows_v, cols_v, buf_v, zbuf, acc_sh, *sems):
    c = lax.axis_index("c")
    s = lax.axis_index("s")
    base, ngrp = _split(c, s)
    for r in range(32):
        for j in range(_H // 16):
            zbuf[r, pl.ds(j * 16, 16)] = jnp.zeros((16,), jnp.float32)

    def zgrp(k, carry):
        pltpu.sync_copy(zbuf, acc_sh.at[pl.ds(s * _RPT + k * 32, 32)])
        return carry

    lax.fori_loop(0, _RPT // 32, zgrp, 0)
    plsc.subcore_barrier()

    def group(g, carry):
        pltpu.sync_copy(rows_hbm.at[pl.ds(base + g * _GRP, _GRP)], rows_v)
        pltpu.sync_copy(cols_hbm.at[pl.ds(base + g * _GRP, _GRP)], cols_v)
        for b in range(_NBUF):
            pltpu.async_copy(table_hbm.at[rows_v.at[b]], buf_v.at[b], sems[b])

        def inner(t, carry2):
            for b in range(_NBUF):
                i = t * _NBUF + b
                pltpu.make_async_copy(
                    table_hbm.at[rows_v.at[i]], buf_v.at[b], sems[b]).wait()
                pltpu.sync_copy(buf_v.at[b], acc_sh.at[cols_v.at[i]],
                                add=True)

                @pl.when(i + _NBUF < _GRP)
                def _():
                    pltpu.async_copy(
                        table_hbm.at[rows_v.at[i + _NBUF]], buf_v.at[b],
                        sems[b])
            return carry2

        lax.fori_loop(0, _GRP // _NBUF, inner, 0)
        return carry

    lax.fori_loop(0, ngrp, group, 0)
    plsc.subcore_barrier()
    pltpu.sync_copy(acc_sh.at[pl.ds(s * _RPT, _RPT)],
                    out_hbm.at[c, pl.ds(s * _RPT, _RPT)])


_mm = lambda a, w: lax.dot_general(a, w, (((1,), (1,)), ((), ())),
                                   preferred_element_type=jnp.float32)


def _tc_first_body(x_ref, w_ref, b_ref, d0_ref, d1_ref, o_ref):
    dis = lax.rsqrt(d0_ref[...] + d1_ref[...] + 1.0)
    h = _mm(x_ref[...], w_ref[...]) + b_ref[...]
    o_ref[...] = h * dis


_tc_first = pl.pallas_call(
    _tc_first_body,
    out_shape=jax.ShapeDtypeStruct((_N, _H), jnp.float32),
)


def _tc_mid_body(s_ref, hp_ref, d0_ref, d1_ref, w_ref, b_ref, o_ref):
    dis = lax.rsqrt(d0_ref[...] + d1_ref[...] + 1.0)
    out = (s_ref[0, :_N, :] + s_ref[1, :_N, :] + hp_ref[...]) * dis
    mu = jnp.mean(out, axis=0, keepdims=True)
    ctr = out - mu
    var = jnp.mean(ctr * ctr, axis=0, keepdims=True)
    a = jnp.maximum(ctr * lax.rsqrt(var + _EPS), 0.0)
    o_ref[...] = (_mm(a, w_ref[...]) + b_ref[...]) * dis


_tc_mid = pl.pallas_call(
    _tc_mid_body,
    out_shape=jax.ShapeDtypeStruct((_N, _H), jnp.float32),
)


def _tc_final_body(s_ref, hp_ref, d0_ref, d1_ref, batch_ref, wl_ref, bl_ref,
                   o_ref):
    dis = lax.rsqrt(d0_ref[...] + d1_ref[...] + 1.0)
    out5 = (s_ref[0, :_N, :] + s_ref[1, :_N, :] + hp_ref[...]) * dis
    gids = lax.broadcasted_iota(jnp.int32, (_G, _N), 0)
    onehot = jnp.where(gids == batch_ref[...], 1.0, 0.0)
    sums = lax.dot_general(onehot, out5, (((1,), (0,)), ((), ())),
                           preferred_element_type=jnp.float32)
    counts = jnp.sum(onehot, axis=1, keepdims=True)
    pooled = sums / jnp.maximum(counts, 1.0)
    o_ref[...] = _mm(pooled, wl_ref[...]) + bl_ref[...]


_tc_final = pl.pallas_call(
    _tc_final_body,
    out_shape=jax.ShapeDtypeStruct((_G, _C), jnp.float32),
)


def kernel(x, edge_index, batch, W1, b1, W2, b2, W3, b3, W4, b4, W5, b5,
           Wl, bl):
    rows_r = edge_index[0].reshape(_RCH, _CHUNK)
    cols_r = edge_index[1].reshape(_RCH, _CHUNK)
    zeros1 = jnp.zeros((_NPAD,), jnp.float32)

    degp = _sc_degree(cols_r, zeros1)
    d0 = degp[0, :_N].reshape(_N, 1)
    d1 = degp[1, :_N].reshape(_N, 1)

    hp = _tc_first(x, W1, b1.reshape(1, _H), d0, d1)
    for (W, b) in ((W2, b2), (W3, b3), (W4, b4)):
        S = _sc_scatter(rows_r, cols_r, hp)
        hp = _tc_mid(S, hp, d0, d1, W, b.reshape(1, _H))
    S = _sc_scatter(rows_r, cols_r, hp)
    hp = _tc_mid(S, hp, d0, d1, W5, b5.reshape(1, _H))
    S = _sc_scatter(rows_r, cols_r, hp)
    return _tc_final(S, hp, d0, d1, batch.reshape(1, _N), Wl,
                     bl.reshape(1, _C))


# speedup vs baseline: 1.1382x; 1.1382x over previous
import functools

import jax
import jax.numpy as jnp
from jax import lax
from jax.experimental import pallas as pl
from jax.experimental.pallas import tpu as pltpu
from jax.experimental.pallas import tpu_sc as plsc

_N = 10000
_D = 128
_H = 128
_C = 10
_G = 64
_E = 320000
_EPS = 1e-5

_NC = 2
_NS = 16
_NW = _NC * _NS
_CHUNK = 64
_NBUF = 4
_ACH = 160
_GRP = 40
_RCH = _E // _CHUNK

_NPAD = 10240
_RPT = _NPAD // _NS


_sc_mesh = dict(core_axis_name="c", subcore_axis_name="s")


def _split(c, s):
    base = (c * _NS + s) * _ACH
    ntail = (_RCH - (_NW - 1) * _ACH) // _GRP
    ngrp = jnp.where((c == 1) & (s == _NS - 1), ntail, _ACH // _GRP)
    return base, ngrp


@functools.partial(
    pl.kernel,
    out_type=jax.ShapeDtypeStruct((_NC, _NPAD), jnp.float32),
    mesh=plsc.VectorSubcoreMesh(**_sc_mesh),
    scratch_types=[
        pltpu.VMEM((_GRP, _CHUNK), jnp.int32),
        pltpu.VMEM((_CHUNK,), jnp.float32),
        pltpu.VMEM_SHARED((_NPAD,), jnp.float32),
    ],
)
def _sc_degree(cols_hbm, zeros1_hbm, out_hbm, cols_v, ones_v, deg_sh):
    c = lax.axis_index("c")
    s = lax.axis_index("s")
    base, ngrp = _split(c, s)
    for j in range(_CHUNK // 16):
        ones_v[pl.ds(j * 16, 16)] = jnp.full((16,), 1.0, jnp.float32)
    pltpu.sync_copy(zeros1_hbm.at[pl.ds(s * _RPT, _RPT)],
                    deg_sh.at[pl.ds(s * _RPT, _RPT)])
    plsc.subcore_barrier()

    def group(g, carry):
        pltpu.sync_copy(cols_hbm.at[pl.ds(base + g * _GRP, _GRP)], cols_v)

        def body(i, carry2):
            pltpu.sync_copy(ones_v, deg_sh.at[cols_v.at[i]], add=True)
            return carry2

        lax.fori_loop(0, _GRP, body, 0)
        return carry

    lax.fori_loop(0, ngrp, group, 0)
    plsc.subcore_barrier()
    pltpu.sync_copy(deg_sh.at[pl.ds(s * _RPT, _RPT)],
                    out_hbm.at[c, pl.ds(s * _RPT, _RPT)])


@functools.partial(
    pl.kernel,
    out_type=jax.ShapeDtypeStruct((_NC, _NPAD, _H), jnp.float32),
    mesh=plsc.VectorSubcoreMesh(**_sc_mesh),
    scratch_types=[
        pltpu.VMEM((_GRP, _CHUNK), jnp.int32),
        pltpu.VMEM((_GRP, _CHUNK), jnp.int32),
        pltpu.VMEM((_NBUF, _CHUNK, _H), jnp.float32),
        pltpu.VMEM((32, _H), jnp.float32),
        pltpu.VMEM_SHARED((_NPAD, _H), jnp.float32),
    ] + [pltpu.SemaphoreType.DMA] * 4,
)
def _sc_scatter(rows_hbm, cols_hbm, table_hbm, out_hbm,
                rows_v, cols_v, buf_v, zbuf, acc_sh, *sems):
    c = lax.axis_index("c")
    s = lax.axis_index("s")
    base, ngrp = _split(c, s)
    for r in range(32):
        for j in range(_H // 16):
            zbuf[r, pl.ds(j * 16, 16)] = jnp.zeros((16,), jnp.float32)

    def zgrp(k, carry):
        pltpu.sync_copy(zbuf, acc_sh.at[pl.ds(s * _RPT + k * 32, 32)])
        return carry

    lax.fori_loop(0, _RPT // 32, zgrp, 0)
    plsc.subcore_barrier()

    def group(g, carry):
        pltpu.sync_copy(rows_hbm.at[pl.ds(base + g * _GRP, _GRP)], rows_v)
        pltpu.sync_copy(cols_hbm.at[pl.ds(base + g * _GRP, _GRP)], cols_v)
        for b in range(_NBUF):
            pltpu.async_copy(table_hbm.at[rows_v.at[b]], buf_v.at[b], sems[b])

        def inner(t, carry2):
            for b in range(_NBUF):
                i = t * _NBUF + b
                pltpu.make_async_copy(
                    table_hbm.at[rows_v.at[i]], buf_v.at[b], sems[b]).wait()
                pltpu.sync_copy(buf_v.at[b], acc_sh.at[cols_v.at[i]],
                                add=True)

                @pl.when(i + _NBUF < _GRP)
                def _():
                    pltpu.async_copy(
                        table_hbm.at[rows_v.at[i + _NBUF]], buf_v.at[b],
                        sems[b])
            return carry2

        lax.fori_loop(0, _GRP // _NBUF, inner, 0)
        return carry

    lax.fori_loop(0, ngrp, group, 0)
    plsc.subcore_barrier()
    pltpu.sync_copy(acc_sh.at[pl.ds(s * _RPT, _RPT)],
                    out_hbm.at[c, pl.ds(s * _RPT, _RPT)])


_mm = lambda a, w: lax.dot_general(a, w, (((1,), (1,)), ((), ())),
                                   preferred_element_type=jnp.float32)


def _tc_first_body(x_ref, w_ref, b_ref, d0_ref, d1_ref, o_ref):
    dis = lax.rsqrt(d0_ref[...] + d1_ref[...] + 1.0)
    h = _mm(x_ref[...], w_ref[...]) + b_ref[...]
    o_ref[...] = h * dis


_tc_first = pl.pallas_call(
    _tc_first_body,
    out_shape=jax.ShapeDtypeStruct((_N, _H), jnp.float32),
)


def _tc_mid_body(s_ref, hp_ref, d0_ref, d1_ref, w_ref, b_ref, o_ref):
    dis = lax.rsqrt(d0_ref[...] + d1_ref[...] + 1.0)
    out = (s_ref[0, :_N, :] + s_ref[1, :_N, :] + hp_ref[...]) * dis
    mu = jnp.mean(out, axis=0, keepdims=True)
    ctr = out - mu
    var = jnp.mean(ctr * ctr, axis=0, keepdims=True)
    a = jnp.maximum(ctr * lax.rsqrt(var + _EPS), 0.0)
    o_ref[...] = (_mm(a, w_ref[...]) + b_ref[...]) * dis


_tc_mid = pl.pallas_call(
    _tc_mid_body,
    out_shape=jax.ShapeDtypeStruct((_N, _H), jnp.float32),
)


def _tc_final_body(s_ref, hp_ref, d0_ref, d1_ref, batch_ref, wl_ref, bl_ref,
                   o_ref):
    dis = lax.rsqrt(d0_ref[...] + d1_ref[...] + 1.0)
    out5 = (s_ref[0, :_N, :] + s_ref[1, :_N, :] + hp_ref[...]) * dis
    gids = lax.broadcasted_iota(jnp.int32, (_G, _N), 0)
    onehot = jnp.where(gids == batch_ref[...], 1.0, 0.0)
    sums = lax.dot_general(onehot, out5, (((1,), (0,)), ((), ())),
                           preferred_element_type=jnp.float32)
    counts = jnp.sum(onehot, axis=1, keepdims=True)
    pooled = sums / jnp.maximum(counts, 1.0)
    o_ref[...] = _mm(pooled, wl_ref[...]) + bl_ref[...]


_tc_final = pl.pallas_call(
    _tc_final_body,
    out_shape=jax.ShapeDtypeStruct((_G, _C), jnp.float32),
)


def kernel(x, edge_index, batch, W1, b1, W2, b2, W3, b3, W4, b4, W5, b5,
           Wl, bl):
    rows_r = edge_index[0].reshape(_RCH, _CHUNK)
    cols_r = edge_index[1].reshape(_RCH, _CHUNK)
    zeros1 = jnp.zeros((_NPAD,), jnp.float32)

    degp = _sc_degree(cols_r, zeros1)
    d0 = degp[0, :_N].reshape(_N, 1)
    d1 = degp[1, :_N].reshape(_N, 1)

    hp = _tc_first(x, W1, b1.reshape(1, _H), d0, d1)
    for (W, b) in ((W2, b2), (W3, b3), (W4, b4)):
        S = _sc_scatter(rows_r, cols_r, hp)
        hp = _tc_mid(S, hp, d0, d1, W, b.reshape(1, _H))
    S = _sc_scatter(rows_r, cols_r, hp)
    hp = _tc_mid(S, hp, d0, d1, W5, b5.reshape(1, _H))
    S = _sc_scatter(rows_r, cols_r, hp)
    return _tc_final(S, hp, d0, d1, batch.reshape(1, _N), Wl,
                     bl.reshape(1, _C))


# speedup vs baseline: 1.1558x; 1.0155x over previous
import functools

import jax
import jax.numpy as jnp
from jax import lax
from jax.experimental import pallas as pl
from jax.experimental.pallas import tpu as pltpu
from jax.experimental.pallas import tpu_sc as plsc

_N = 10000
_D = 128
_H = 128
_C = 10
_G = 64
_E = 320000
_EPS = 1e-5

_NC = 2
_NS = 16
_NW = _NC * _NS
_CHUNK = 64
_NBUF = 4
_ACH = 160
_GRP = 40
_RCH = _E // _CHUNK

_NPAD = 10240
_RPT = _NPAD // _NS


_sc_mesh = dict(core_axis_name="c", subcore_axis_name="s")


def _split(c, s):
    base = (c * _NS + s) * _ACH
    ntail = (_RCH - (_NW - 1) * _ACH) // _GRP
    ngrp = jnp.where((c == 1) & (s == _NS - 1), ntail, _ACH // _GRP)
    return base, ngrp


@functools.partial(
    pl.kernel,
    out_type=jax.ShapeDtypeStruct((_NC, _NPAD), jnp.float32),
    mesh=plsc.VectorSubcoreMesh(**_sc_mesh),
    scratch_types=[
        pltpu.VMEM((_GRP, _CHUNK), jnp.int32),
        pltpu.VMEM((_CHUNK,), jnp.float32),
        pltpu.VMEM_SHARED((_NPAD,), jnp.float32),
        pltpu.SemaphoreType.DMA,
    ],
)
def _sc_degree(cols_hbm, zeros1_hbm, out_hbm, cols_v, ones_v, deg_sh, sem):
    c = lax.axis_index("c")
    s = lax.axis_index("s")
    base, ngrp = _split(c, s)
    for j in range(_CHUNK // 16):
        ones_v[pl.ds(j * 16, 16)] = jnp.full((16,), 1.0, jnp.float32)
    pltpu.sync_copy(zeros1_hbm.at[pl.ds(s * _RPT, _RPT)],
                    deg_sh.at[pl.ds(s * _RPT, _RPT)])
    plsc.subcore_barrier()

    def group(g, carry):
        pltpu.sync_copy(cols_hbm.at[pl.ds(base + g * _GRP, _GRP)], cols_v)

        def fire(i, carry2):
            pltpu.async_copy(ones_v, deg_sh.at[cols_v.at[i]], sem, add=True)
            return carry2

        lax.fori_loop(0, _GRP, fire, 0)

        def drain(i, carry2):
            pltpu.make_async_copy(ones_v, deg_sh.at[cols_v.at[i]],
                                  sem).wait()
            return carry2

        lax.fori_loop(0, _GRP, drain, 0)
        return carry

    lax.fori_loop(0, ngrp, group, 0)
    plsc.subcore_barrier()
    pltpu.sync_copy(deg_sh.at[pl.ds(s * _RPT, _RPT)],
                    out_hbm.at[c, pl.ds(s * _RPT, _RPT)])


@functools.partial(
    pl.kernel,
    out_type=jax.ShapeDtypeStruct((_NC, _NPAD, _H), jnp.float32),
    mesh=plsc.VectorSubcoreMesh(**_sc_mesh),
    scratch_types=[
        pltpu.VMEM((_GRP, _CHUNK), jnp.int32),
        pltpu.VMEM((_GRP, _CHUNK), jnp.int32),
        pltpu.VMEM((_NBUF, _CHUNK, _H), jnp.float32),
        pltpu.VMEM((32, _H), jnp.float32),
        pltpu.VMEM_SHARED((_NPAD, _H), jnp.float32),
    ] + [pltpu.SemaphoreType.DMA] * 4,
)
def _sc_scatter(rows_hbm, cols_hbm, table_hbm, out_hbm,
                rows_v, cols_v, buf_v, zbuf, acc_sh, *sems):
    c = lax.axis_index("c")
    s = lax.axis_index("s")
    base, ngrp = _split(c, s)
    for r in range(32):
        for j in range(_H // 16):
            zbuf[r, pl.ds(j * 16, 16)] = jnp.zeros((16,), jnp.float32)

    def zgrp(k, carry):
        pltpu.sync_copy(zbuf, acc_sh.at[pl.ds(s * _RPT + k * 32, 32)])
        return carry

    lax.fori_loop(0, _RPT // 32, zgrp, 0)
    plsc.subcore_barrier()

    def group(g, carry):
        pltpu.sync_copy(rows_hbm.at[pl.ds(base + g * _GRP, _GRP)], rows_v)
        pltpu.sync_copy(cols_hbm.at[pl.ds(base + g * _GRP, _GRP)], cols_v)
        for b in range(_NBUF):
            pltpu.async_copy(table_hbm.at[rows_v.at[b]], buf_v.at[b], sems[b])

        def inner(t, carry2):
            for b in range(_NBUF):
                i = t * _NBUF + b
                pltpu.make_async_copy(
                    table_hbm.at[rows_v.at[i]], buf_v.at[b], sems[b]).wait()
                pltpu.sync_copy(buf_v.at[b], acc_sh.at[cols_v.at[i]],
                                add=True)

                @pl.when(i + _NBUF < _GRP)
                def _():
                    pltpu.async_copy(
                        table_hbm.at[rows_v.at[i + _NBUF]], buf_v.at[b],
                        sems[b])
            return carry2

        lax.fori_loop(0, _GRP // _NBUF, inner, 0)
        return carry

    lax.fori_loop(0, ngrp, group, 0)
    plsc.subcore_barrier()
    pltpu.sync_copy(acc_sh.at[pl.ds(s * _RPT, _RPT)],
                    out_hbm.at[c, pl.ds(s * _RPT, _RPT)])


_mm = lambda a, w: lax.dot_general(a, w, (((1,), (1,)), ((), ())),
                                   preferred_element_type=jnp.float32)


def _tc_first_body(x_ref, w_ref, b_ref, d0_ref, d1_ref, o_ref):
    dis = lax.rsqrt(d0_ref[...] + d1_ref[...] + 1.0)
    h = _mm(x_ref[...], w_ref[...]) + b_ref[...]
    o_ref[...] = h * dis


_tc_first = pl.pallas_call(
    _tc_first_body,
    out_shape=jax.ShapeDtypeStruct((_N, _H), jnp.float32),
)


def _tc_mid_body(s_ref, hp_ref, d0_ref, d1_ref, w_ref, b_ref, o_ref):
    dis = lax.rsqrt(d0_ref[...] + d1_ref[...] + 1.0)
    out = (s_ref[0, :_N, :] + s_ref[1, :_N, :] + hp_ref[...]) * dis
    mu = jnp.mean(out, axis=0, keepdims=True)
    ctr = out - mu
    var = jnp.mean(ctr * ctr, axis=0, keepdims=True)
    a = jnp.maximum(ctr * lax.rsqrt(var + _EPS), 0.0)
    o_ref[...] = (_mm(a, w_ref[...]) + b_ref[...]) * dis


_tc_mid = pl.pallas_call(
    _tc_mid_body,
    out_shape=jax.ShapeDtypeStruct((_N, _H), jnp.float32),
)


def _tc_final_body(s_ref, hp_ref, d0_ref, d1_ref, batch_ref, wl_ref, bl_ref,
                   o_ref):
    dis = lax.rsqrt(d0_ref[...] + d1_ref[...] + 1.0)
    out5 = (s_ref[0, :_N, :] + s_ref[1, :_N, :] + hp_ref[...]) * dis
    gids = lax.broadcasted_iota(jnp.int32, (_G, _N), 0)
    onehot = jnp.where(gids == batch_ref[...], 1.0, 0.0)
    sums = lax.dot_general(onehot, out5, (((1,), (0,)), ((), ())),
                           preferred_element_type=jnp.float32)
    counts = jnp.sum(onehot, axis=1, keepdims=True)
    pooled = sums / jnp.maximum(counts, 1.0)
    o_ref[...] = _mm(pooled, wl_ref[...]) + bl_ref[...]


_tc_final = pl.pallas_call(
    _tc_final_body,
    out_shape=jax.ShapeDtypeStruct((_G, _C), jnp.float32),
)


def kernel(x, edge_index, batch, W1, b1, W2, b2, W3, b3, W4, b4, W5, b5,
           Wl, bl):
    rows_r = edge_index[0].reshape(_RCH, _CHUNK)
    cols_r = edge_index[1].reshape(_RCH, _CHUNK)
    zeros1 = jnp.zeros((_NPAD,), jnp.float32)

    degp = _sc_degree(cols_r, zeros1)
    d0 = degp[0, :_N].reshape(_N, 1)
    d1 = degp[1, :_N].reshape(_N, 1)

    hp = _tc_first(x, W1, b1.reshape(1, _H), d0, d1)
    for (W, b) in ((W2, b2), (W3, b3), (W4, b4)):
        S = _sc_scatter(rows_r, cols_r, hp)
        hp = _tc_mid(S, hp, d0, d1, W, b.reshape(1, _H))
    S = _sc_scatter(rows_r, cols_r, hp)
    hp = _tc_mid(S, hp, d0, d1, W5, b5.reshape(1, _H))
    S = _sc_scatter(rows_r, cols_r, hp)
    return _tc_final(S, hp, d0, d1, batch.reshape(1, _N), Wl,
                     bl.reshape(1, _C))
